# baseline (speedup 1.0000x reference)
import functools

import jax
import jax.numpy as jnp
import numpy as np
from jax import lax
from jax.experimental import pallas as pl
from jax.experimental.pallas import tpu as pltpu
from jax.experimental.pallas import tpu_sc as plsc

T = 64
RCHUNK = 128


def _route_body(x_ref, wg_ref, pos_ref, bexp_ref, vcnt_ref, lb_ref):
    N = x_ref.shape[0]
    E = wg_ref.shape[1]

    logits = jax.lax.dot_general(
        x_ref[...], wg_ref[...], (((1,), (0,)), ((), ())),
        preferred_element_type=jnp.float32)
    m = jnp.max(logits, axis=1, keepdims=True)
    iota_ne = jax.lax.broadcasted_iota(jnp.int32, (N, E), 1)
    eidx = jnp.min(jnp.where(logits == m, iota_ne, E), axis=1)
    onehot = (iota_ne == eidx[:, None]).astype(jnp.float32)

    counts = jnp.sum(onehot, axis=0)
    frac = counts * (1.0 / N)
    lb_ref[...] = jnp.broadcast_to(E * jnp.sum(frac * frac) * 0.1, (1, 1))

    iota_r = jax.lax.broadcasted_iota(jnp.int32, (RCHUNK, RCHUNK), 0)
    iota_c = jax.lax.broadcasted_iota(jnp.int32, (RCHUNK, RCHUNK), 1)
    lt = (iota_c < iota_r).astype(jnp.bfloat16)
    oh_b = onehot.astype(jnp.bfloat16)
    base = jnp.zeros((1, E), jnp.float32)
    rank_rows = []
    for c in range(N // RCHUNK):
        oc = oh_b[c * RCHUNK:(c + 1) * RCHUNK]
        local = jax.lax.dot_general(
            lt, oc, (((1,), (0,)), ((), ())),
            preferred_element_type=jnp.float32)
        rank_rows.append(local + base)
        base = base + jnp.sum(oc.astype(jnp.float32), axis=0, keepdims=True)
    rank_all = jnp.concatenate(rank_rows, axis=0)
    rank = jnp.sum(rank_all * onehot, axis=1)

    nblk = jnp.floor((counts + (T - 1)) * (1.0 / T))
    iota_ee_r = jax.lax.broadcasted_iota(jnp.int32, (E, E), 0)
    iota_ee_c = jax.lax.broadcasted_iota(jnp.int32, (E, E), 1)
    ltE = (iota_ee_c < iota_ee_r).astype(jnp.float32)
    bstart = jnp.sum(ltE * nblk[None, :], axis=1)
    pad_off = bstart * float(T)

    pos = jnp.sum(onehot * pad_off[None, :], axis=1) + rank
    pos_ref[...] = pos.astype(jnp.int32)

    NBP = bexp_ref.shape[0]
    iota_be_b = jax.lax.broadcasted_iota(
        jnp.int32, (NBP, E), 0).astype(jnp.float32)
    iota_be_e = jax.lax.broadcasted_iota(
        jnp.int32, (NBP, E), 1).astype(jnp.float32)
    inside = ((iota_be_b >= bstart[None, :])
              & (iota_be_b < (bstart + nblk)[None, :])).astype(jnp.float32)
    bexp_ref[...] = jnp.sum(inside * iota_be_e, axis=1).astype(jnp.int32)
    cnt_blk = jnp.clip(
        counts[None, :] - (iota_be_b - bstart[None, :]) * float(T),
        0.0, float(T))
    vcnt_ref[...] = jnp.sum(inside * cnt_blk, axis=1).astype(jnp.int32)


def _route(xf, Wg, nbp):
    N, D = xf.shape
    E = Wg.shape[1]
    return pl.pallas_call(
        _route_body,
        in_specs=[
            pl.BlockSpec((N, D), lambda: (0, 0)),
            pl.BlockSpec((D, E), lambda: (0, 0)),
        ],
        out_specs=[
            pl.BlockSpec((N,), lambda: (0,)),
            pl.BlockSpec((nbp,), lambda: (0,)),
            pl.BlockSpec((nbp,), lambda: (0,)),
            pl.BlockSpec((1, 1), lambda: (0, 0)),
        ],
        out_shape=[
            jax.ShapeDtypeStruct((N,), jnp.int32),
            jax.ShapeDtypeStruct((nbp,), jnp.int32),
            jax.ShapeDtypeStruct((nbp,), jnp.int32),
            jax.ShapeDtypeStruct((1, 1), jnp.float32),
        ],
    )(xf, Wg)


def _fused_body(bexp_ref, vcnt_ref, xpad_ref, wdw_ref, bdw_ref,
                wup_ref, bup_ref, outpad_ref):
    b = pl.program_id(0)

    e = bexp_ref[b]
    w = wdw_ref[e].astype(jnp.bfloat16)
    het = jax.lax.dot_general(
        w, xpad_ref[...].astype(jnp.bfloat16), (((1,), (1,)), ((), ())),
        preferred_element_type=jnp.float32)
    het = het + bdw_ref[e]
    actt = 0.5 * het * (1.0 + jnp.tanh(
        np.sqrt(2.0 / np.pi) * (het + 0.044715 * het * het * het)))
    act = jnp.transpose(actt)
    outpad_ref[...] = jax.lax.dot_general(
        act, wup_ref[...], (((1,), (1,)), ((), ())),
        preferred_element_type=jnp.float32) + bup_ref[...][None, :]


def _fused(xpad, Wdw, bdw3, Wup, bup, bexp, vcnt, nb):
    NPAD, D = xpad.shape
    E, DOWN, _ = Wdw.shape
    return pl.pallas_call(
        _fused_body,
        grid_spec=pltpu.PrefetchScalarGridSpec(
            num_scalar_prefetch=2,
            grid=(nb,),
            in_specs=[
                pl.BlockSpec((T, D), lambda b, be, vc: (b, 0)),
                pl.BlockSpec((E, DOWN, D), lambda b, be, vc: (0, 0, 0)),
                pl.BlockSpec((E, DOWN, 1), lambda b, be, vc: (0, 0, 0)),
                pl.BlockSpec((D, DOWN), lambda b, be, vc: (0, 0)),
                pl.BlockSpec((D,), lambda b, be, vc: (0,)),
            ],
            out_specs=pl.BlockSpec((T, D), lambda b, be, vc: (b, 0)),
        ),
        out_shape=jax.ShapeDtypeStruct((NPAD, D), jnp.float32),
        compiler_params=pltpu.CompilerParams(
            dimension_semantics=("arbitrary",)),
    )(bexp, vcnt, xpad, Wdw, bdw3, Wup, bup)


def _sc_gather_rows(outpad, pos2d):
    NPAD, D = outpad.shape
    NW, CH = pos2d.shape
    N = NW * CH
    mesh = plsc.VectorSubcoreMesh(core_axis_name="c", subcore_axis_name="s")

    @functools.partial(
        pl.kernel, mesh=mesh,
        out_type=jax.ShapeDtypeStruct((N, D), jnp.float32),
        scratch_types=[
            pltpu.VMEM((CH,), jnp.int32),
            pltpu.VMEM((CH, D), jnp.float32),
            pltpu.SemaphoreType.DMA,
        ],
    )
    def k(outpad_hbm, pos_hbm, out_hbm, idx_v, rows_v, sem):
        nc = 2
        wid = lax.axis_index("s") * nc + lax.axis_index("c")
        pltpu.sync_copy(pos_hbm.at[wid], idx_v)
        pltpu.async_copy(outpad_hbm.at[idx_v], rows_v, sem).wait()
        pltpu.sync_copy(rows_v, out_hbm.at[pl.ds(wid * CH, CH)])

    return k(outpad, pos2d)


def _sc_scatter_rows(xf, pos2d, npad):
    N, D = xf.shape
    NW, CH = pos2d.shape
    mesh = plsc.VectorSubcoreMesh(core_axis_name="c", subcore_axis_name="s")

    @functools.partial(
        pl.kernel, mesh=mesh,
        out_type=jax.ShapeDtypeStruct((npad, D), jnp.float32),
        scratch_types=[
            pltpu.VMEM((CH,), jnp.int32),
            pltpu.VMEM((CH, D), jnp.float32),
            pltpu.SemaphoreType.DMA,
        ],
    )
    def k(x_hbm, pos_hbm, xpad_hbm, idx_v, rows_v, sem):
        nc = 2
        wid = lax.axis_index("s") * nc + lax.axis_index("c")
        pltpu.sync_copy(pos_hbm.at[wid], idx_v)
        pltpu.sync_copy(x_hbm.at[pl.ds(wid * CH, CH)], rows_v)
        pltpu.async_copy(rows_v, xpad_hbm.at[idx_v], sem).wait()

    return k(xf, pos2d)


def kernel(x, Wg, Wdw, bdw, Wup, bup):
    B, S, D = x.shape
    E, DOWN, _ = Wdw.shape
    N = B * S
    xf = x.reshape(N, D)

    NB = N // T + E
    NBP = -(-NB // 128) * 128
    NPAD = NB * T
    NW = 32

    pos, bexp, vcnt, lb = _route(xf, Wg, NBP)
    pos2d = pos.reshape(NW, N // NW)
    xpad = _sc_scatter_rows(xf, pos2d, NPAD)
    outpad = _fused(xpad, Wdw, bdw.reshape(E, DOWN, 1), Wup, bup,
                    bexp, vcnt, NB)
    out = _sc_gather_rows(outpad, pos2d)
    return out.reshape(B, S, D), lb.reshape(())


# speedup vs baseline: 1.5422x; 1.5422x over previous
import jax
import jax.numpy as jnp
import numpy as np
from jax.experimental import pallas as pl
from jax.experimental.pallas import tpu as pltpu


def _moe_body(x_ref, wg_ref, wdw_ref, bdw_ref, wup_ref, bup_ref,
              out_ref, lb_ref, eidx_scr, ht_scr, xt_scr):
    e = pl.program_id(0)
    n_e = pl.num_programs(0)
    N = x_ref.shape[0]
    E = wg_ref.shape[1]

    @pl.when(e == 0)
    def _gate():
        logits = jax.lax.dot_general(
            x_ref[...], wg_ref[...], (((1,), (0,)), ((), ())),
            preferred_element_type=jnp.float32)
        m = jnp.max(logits, axis=1, keepdims=True)
        iota_ne = jax.lax.broadcasted_iota(jnp.int32, (N, E), 1)
        idx = jnp.min(jnp.where(logits == m, iota_ne, E), axis=1)
        eidx_scr[...] = idx[None, :]
        onehot = (iota_ne == idx[:, None]).astype(jnp.float32)
        counts = jnp.sum(onehot, axis=0)
        frac = counts * (1.0 / N)
        lb_ref[...] = jnp.broadcast_to(E * jnp.sum(frac * frac) * 0.1, (1, 1))
        xt_scr[...] = jnp.transpose(x_ref[...]).astype(jnp.bfloat16)

    w = wdw_ref[0].astype(jnp.bfloat16)
    het = jax.lax.dot_general(
        w, xt_scr[...], (((1,), (0,)), ((), ())),
        preferred_element_type=jnp.float32)
    het = het + bdw_ref[0]
    sel = eidx_scr[...] == e
    prev = jnp.where(e == 0, jnp.zeros_like(het), ht_scr[...])
    ht_scr[...] = jnp.where(sel, het, prev)

    @pl.when(e == n_e - 1)
    def _up():
        ht = ht_scr[...]
        actt = 0.5 * ht * (1.0 + jnp.tanh(
            np.sqrt(2.0 / np.pi) * (ht + 0.044715 * ht * ht * ht)))
        act = jnp.transpose(actt)
        out_ref[...] = jax.lax.dot_general(
            act, wup_ref[...], (((1,), (1,)), ((), ())),
            preferred_element_type=jnp.float32) + bup_ref[...][None, :]


def kernel(x, Wg, Wdw, bdw, Wup, bup):
    B, S, D = x.shape
    E, DOWN, _ = Wdw.shape
    N = B * S
    xf = x.reshape(N, D)

    out, lb = pl.pallas_call(
        _moe_body,
        grid=(E,),
        in_specs=[
            pl.BlockSpec((N, D), lambda e: (0, 0)),
            pl.BlockSpec((D, E), lambda e: (0, 0)),
            pl.BlockSpec((1, DOWN, D), lambda e: (e, 0, 0)),
            pl.BlockSpec((1, DOWN, 1), lambda e: (e, 0, 0)),
            pl.BlockSpec((D, DOWN), lambda e: (0, 0)),
            pl.BlockSpec((D,), lambda e: (0,)),
        ],
        out_specs=[
            pl.BlockSpec((N, D), lambda e: (0, 0)),
            pl.BlockSpec((1, 1), lambda e: (0, 0)),
        ],
        out_shape=[
            jax.ShapeDtypeStruct((N, D), jnp.float32),
            jax.ShapeDtypeStruct((1, 1), jnp.float32),
        ],
        scratch_shapes=[
            pltpu.VMEM((1, N), jnp.int32),
            pltpu.VMEM((DOWN, N), jnp.float32),
            pltpu.VMEM((D, N), jnp.bfloat16),
        ],
        compiler_params=pltpu.CompilerParams(
            dimension_semantics=("arbitrary",)),
    )(xf, Wg, Wdw, bdw.reshape(E, DOWN, 1), Wup, bup)

    return out.reshape(B, S, D), lb.reshape(())
